# baseline (speedup 1.0000x reference)
import functools

import jax
import jax.numpy as jnp
from jax import lax
from jax.experimental import pallas as pl
from jax.experimental.pallas import tpu as pltpu
from jax.experimental.pallas import tpu_sc as plsc

B = 4
P = 5
S = 2048
D = 1024
SLAB = S * D
NW = 32
PER_W = SLAB // NW
CHUNK = 8192
ROWS = CHUNK // D
NCHUNK = PER_W // CHUNK
LANES = 16


def _body(probes_hbm, scores_hbm, rexp_hbm, out_hbm,
          scores_v, rexp_v, x0, x1, ob):
    cid = lax.axis_index("c")
    sid = lax.axis_index("s")
    wid = sid * 2 + cid
    base = wid * PER_W

    pltpu.sync_copy(scores_hbm, scores_v)
    pltpu.sync_copy(rexp_hbm, rexp_v)

    sc_lo = scores_v[pl.ds(0, LANES)]
    sc_hi = scores_v[pl.ds(LANES, LANES)]

    for b in range(B):
        sv = [(sc_lo[P * b + i] if P * b + i < LANES else sc_hi[P * b + i - LANES])
              for i in range(P)]
        bv = sv[0]
        bi = jnp.int32(0)
        for i in range(1, P):
            better = sv[i] > bv
            bv = jnp.where(better, sv[i], bv)
            bi = jnp.where(better, jnp.int32(i), bi)
        b2v = jnp.float32(-jnp.inf)
        b2i = jnp.int32(0)
        for i in range(P):
            cand = jnp.logical_and(jnp.int32(i) != bi, sv[i] > b2v)
            b2v = jnp.where(cand, sv[i], b2v)
            b2i = jnp.where(cand, jnp.int32(i), b2i)
        delta = jnp.full((LANES,), b2v - bv, dtype=jnp.float32)
        e = jnp.exp(delta)
        w1v = e / (1.0 + e)
        w0v = 1.0 - w1v

        row0 = P * b + bi
        row1 = P * b + b2i

        def chunk_body(c, carry):
            off = base + c * CHUNK
            pltpu.sync_copy(probes_hbm.at[row0, pl.ds(off, CHUNK)], x0)
            pltpu.sync_copy(probes_hbm.at[row1, pl.ds(off, CHUNK)], x1)

            def jbody(j, carry2):
                col = j * LANES
                revs = [rexp_v[pl.ds(p * D + col, LANES)] for p in range(P)]
                for r in range(ROWS):
                    pos = r * D + col
                    v0 = x0[pl.ds(pos, LANES)]
                    v1 = x1[pl.ds(pos, LANES)]
                    m = w0v * v0 + w1v * v1
                    for p in range(P):
                        ob[pl.ds(p * CHUNK + pos, LANES)] = m + revs[p]
                return carry2

            lax.fori_loop(0, D // LANES, jbody, 0)
            for p in range(P):
                pltpu.sync_copy(ob.at[pl.ds(p * CHUNK, CHUNK)],
                                out_hbm.at[P * b + p, pl.ds(off, CHUNK)])
            return carry

        lax.fori_loop(0, NCHUNK, chunk_body, 0)


def kernel(probes, scores, re_expand):
    b, p, s, d = probes.shape
    probes2 = probes.reshape(b * p, s * d)
    scores_pad = jnp.zeros((32,), jnp.float32).at[: b * p].set(
        scores.reshape(-1).astype(jnp.float32))
    rexp = re_expand.reshape(-1).astype(jnp.float32)

    mesh = plsc.VectorSubcoreMesh(core_axis_name="c", subcore_axis_name="s")
    run = functools.partial(
        pl.kernel,
        mesh=mesh,
        out_type=jax.ShapeDtypeStruct((b * p, s * d), jnp.float32),
        scratch_types=[
            pltpu.VMEM((32,), jnp.float32),
            pltpu.VMEM((P * D,), jnp.float32),
            pltpu.VMEM((CHUNK,), jnp.float32),
            pltpu.VMEM((CHUNK,), jnp.float32),
            pltpu.VMEM((P * CHUNK,), jnp.float32),
        ],
    )(_body)
    out2 = run(probes2, scores_pad, rexp)
    return out2.reshape(b, p, s, d)
